# baseline (speedup 1.0000x reference)
import functools

import jax
import jax.numpy as jnp
from jax.experimental import pallas as pl
from jax.experimental.pallas import tpu as pltpu

_BF = 512


def _mega_body(x_ref, xb_ref, gw_ref, w1_ref, w2_ref, comb_ref, m_ref, mdc_ref):
    e = pl.program_id(0)
    f = pl.program_id(1)
    T, E = comb_ref.shape

    @pl.when((e == 0) & (f == 0))
    def _gate():
        x = x_ref[...]
        logits = jax.lax.dot_general(
            x, gw_ref[...], (((1,), (1,)), ((), ())),
            preferred_element_type=jnp.float32)
        idx = jax.lax.broadcasted_iota(jnp.int32, (T, E), 1)
        v1 = jnp.max(logits, axis=1, keepdims=True)
        s1 = jnp.min(jnp.where(logits == v1, idx, E), axis=1, keepdims=True)
        masked = jnp.where(idx == s1, jnp.float32(-1e30), logits)
        v2 = jnp.max(masked, axis=1, keepdims=True)
        s2 = jnp.min(jnp.where(masked == v2, idx, E), axis=1, keepdims=True)
        z = jnp.exp(v2 - v1)
        wa = 1.0 / (1.0 + z)
        wb = z * wa
        oh1 = (idx == s1).astype(jnp.float32)
        oh2 = (idx == s2).astype(jnp.float32)
        comb_ref[...] = wa * oh1 + wb * oh2
        mask = oh1 + oh2
        counts = jnp.sum(mask, axis=0, keepdims=True)
        mdc_ref[...] = mask / jnp.maximum(counts, 1.0)

    xb = xb_ref[...]
    w1e = w1_ref[0]
    h = jax.lax.dot_general(xb, w1e, (((1,), (1,)), ((), ())),
                            preferred_element_type=jnp.float32)
    h = h * (1.0 / (1.0 + jnp.exp(-h)))
    onehot_e = (jax.lax.broadcasted_iota(jnp.int32, (1, E), 1) == e
                ).astype(jnp.float32)
    mcol = jax.lax.dot_general(mdc_ref[...], onehot_e, (((1,), (1,)), ((), ())),
                               preferred_element_type=jnp.float32)
    s = jax.lax.dot_general(mcol, h, (((0,), (0,)), ((), ())),
                            preferred_element_type=jnp.float32)
    part = jax.lax.dot_general(s.astype(jnp.bfloat16), w2_ref[0],
                               (((1,), (1,)), ((), ())),
                               preferred_element_type=jnp.float32)

    @pl.when(f == 0)
    def _init():
        m_ref[0] = part

    @pl.when(f != 0)
    def _acc():
        m_ref[0] = m_ref[0] + part


def _combine_body(comb_ref, m_ref, out_ref):
    out_ref[...] = jax.lax.dot_general(
        comb_ref[...], m_ref[...], (((1,), (0,)), ((), ())),
        preferred_element_type=jnp.float32)


@jax.jit
def kernel(hidden_states, gate_w, W1, W2):
    b, s_len, d = hidden_states.shape
    e_num, dff, _ = W1.shape
    t = b * s_len
    x = hidden_states.reshape(t, d)
    xb = x.astype(jnp.bfloat16)
    w1b = W1.astype(jnp.bfloat16)
    w2b = W2.astype(jnp.bfloat16)
    nf = dff // _BF

    comb, m = pl.pallas_call(
        _mega_body,
        grid=(e_num, nf),
        in_specs=[
            pl.BlockSpec((t, d), lambda e, f: (0, 0)),
            pl.BlockSpec((t, d), lambda e, f: (0, 0)),
            pl.BlockSpec((e_num, d), lambda e, f: (0, 0)),
            pl.BlockSpec((1, _BF, d), lambda e, f: (e, f, 0)),
            pl.BlockSpec((1, d, _BF), lambda e, f: (e, 0, f)),
        ],
        out_specs=[
            pl.BlockSpec((t, e_num), lambda e, f: (0, 0)),
            pl.BlockSpec((1, 1, d), lambda e, f: (e, 0, 0)),
        ],
        out_shape=[
            jax.ShapeDtypeStruct((t, e_num), jnp.float32),
            jax.ShapeDtypeStruct((e_num, 1, d), jnp.float32),
        ],
        scratch_shapes=[pltpu.VMEM((t, e_num), jnp.float32)],
    )(x, xb, gate_w, w1b, w2b)

    out = pl.pallas_call(
        _combine_body,
        out_shape=jax.ShapeDtypeStruct((t, d), jnp.float32),
    )(comb, m.reshape(e_num, d))
    return out.reshape(b, s_len, d)


# speedup vs baseline: 1.4836x; 1.4836x over previous
import functools

import jax
import jax.numpy as jnp
from jax.experimental import pallas as pl
from jax.experimental.pallas import tpu as pltpu

_BF = 512


def _mega_body(x_ref, gw_ref, w1_ref, w2_ref, comb_ref, m_ref, mdc_ref, xb_ref):
    e = pl.program_id(0)
    f = pl.program_id(1)
    T, E = comb_ref.shape

    @pl.when((e == 0) & (f == 0))
    def _gate():
        x = x_ref[...]
        logits = jax.lax.dot_general(
            x, gw_ref[...], (((1,), (1,)), ((), ())),
            preferred_element_type=jnp.float32)
        idx = jax.lax.broadcasted_iota(jnp.int32, (T, E), 1)
        v1 = jnp.max(logits, axis=1, keepdims=True)
        s1 = jnp.min(jnp.where(logits == v1, idx, E), axis=1, keepdims=True)
        masked = jnp.where(idx == s1, jnp.float32(-1e30), logits)
        v2 = jnp.max(masked, axis=1, keepdims=True)
        s2 = jnp.min(jnp.where(masked == v2, idx, E), axis=1, keepdims=True)
        z = jnp.exp(v2 - v1)
        wa = 1.0 / (1.0 + z)
        wb = z * wa
        oh1 = (idx == s1).astype(jnp.float32)
        oh2 = (idx == s2).astype(jnp.float32)
        comb_ref[...] = wa * oh1 + wb * oh2
        mask = oh1 + oh2
        counts = jnp.sum(mask, axis=0, keepdims=True)
        mdc_ref[...] = mask / jnp.maximum(counts, 1.0)
        xb_ref[...] = x.astype(jnp.bfloat16)

    xb = xb_ref[...]
    w1e = w1_ref[0].astype(jnp.bfloat16)
    h = jax.lax.dot_general(xb, w1e, (((1,), (1,)), ((), ())),
                            preferred_element_type=jnp.float32)
    h = h * (1.0 / (1.0 + jnp.exp(-h)))
    onehot_e = (jax.lax.broadcasted_iota(jnp.int32, (1, E), 1) == e
                ).astype(jnp.float32)
    mcol = jax.lax.dot_general(mdc_ref[...], onehot_e, (((1,), (1,)), ((), ())),
                               preferred_element_type=jnp.float32)
    s = jax.lax.dot_general(mcol, h, (((0,), (0,)), ((), ())),
                            preferred_element_type=jnp.float32)
    part = jax.lax.dot_general(s, w2_ref[0], (((1,), (1,)), ((), ())),
                               preferred_element_type=jnp.float32)

    @pl.when(f == 0)
    def _init():
        m_ref[0] = part

    @pl.when(f != 0)
    def _acc():
        m_ref[0] = m_ref[0] + part


def _combine_body(comb_ref, m_ref, out_ref):
    out_ref[...] = jax.lax.dot_general(
        comb_ref[...], m_ref[...], (((1,), (0,)), ((), ())),
        preferred_element_type=jnp.float32)


@jax.jit
def kernel(hidden_states, gate_w, W1, W2):
    b, s_len, d = hidden_states.shape
    e_num, dff, _ = W1.shape
    t = b * s_len
    x = hidden_states.reshape(t, d)
    nf = dff // _BF

    comb, m = pl.pallas_call(
        _mega_body,
        grid=(e_num, nf),
        in_specs=[
            pl.BlockSpec((t, d), lambda e, f: (0, 0)),
            pl.BlockSpec((e_num, d), lambda e, f: (0, 0)),
            pl.BlockSpec((1, _BF, d), lambda e, f: (e, f, 0)),
            pl.BlockSpec((1, d, _BF), lambda e, f: (e, 0, f)),
        ],
        out_specs=[
            pl.BlockSpec((t, e_num), lambda e, f: (0, 0)),
            pl.BlockSpec((1, 1, d), lambda e, f: (e, 0, 0)),
        ],
        out_shape=[
            jax.ShapeDtypeStruct((t, e_num), jnp.float32),
            jax.ShapeDtypeStruct((e_num, 1, d), jnp.float32),
        ],
        scratch_shapes=[pltpu.VMEM((t, e_num), jnp.float32),
                        pltpu.VMEM((t, d), jnp.bfloat16)],
    )(x, gate_w, W1, W2)

    out = pl.pallas_call(
        _combine_body,
        out_shape=jax.ShapeDtypeStruct((t, d), jnp.float32),
    )(comb, m.reshape(e_num, d))
    return out.reshape(b, s_len, d)
